# baseline (speedup 1.0000x reference)
import functools
import math as _math

import jax
import jax.numpy as jnp
from jax import lax
from jax.experimental import pallas as pl
from jax.experimental.pallas import tpu as pltpu
from jax.experimental.pallas import tpu_sc as plsc

_SIZE = 100000
_N = 1024
_SMOOTHING = 0.1
_CONF = 1.0 - _SMOOTHING
_S = _SMOOTHING / (_SIZE - 2)
_C1 = _CONF * _math.log(_CONF) + _SMOOTHING * _math.log(_S)


_BC = 512
_P = 4
_NBC = 49
_NBTOT = _P * _NBC


def _tc_body(t_ref, x_ref, o_ref):
    i = pl.program_id(0)
    k = pl.program_id(1)
    gb = i * _NBC + k
    xb = x_ref[...]
    m = (t_ref[...] != 0).astype(jnp.float32)

    edge = jnp.logical_or(gb == 0, gb == _NBTOT - 1)

    @pl.when(k == 0)
    def _():
        o_ref[...] = jnp.zeros_like(o_ref)

    @pl.when(edge)
    def _():
        col = lax.broadcasted_iota(jnp.int32, xb.shape, 1) + gb * _BC
        valid = jnp.logical_and(col >= 1, col < _SIZE)
        rows = jnp.sum(jnp.where(valid, xb, 0.0), axis=1, keepdims=True)
        part = -_S * (rows * m)
        part = part + jnp.where(gb == 0, _C1 * m, 0.0)
        o_ref[...] += part

    @pl.when(jnp.logical_not(edge))
    def _():
        rows = jnp.sum(xb, axis=1, keepdims=True)
        o_ref[...] += -_S * (rows * m)


def _tc_reduce(x, t2d):
    return pl.pallas_call(
        _tc_body,
        grid=(_P, _NBC),
        in_specs=[
            pl.BlockSpec((_N, 1), lambda i, k: (0, 0)),
            pl.BlockSpec((_N, _BC), lambda i, k: (0, i * _NBC + k)),
        ],
        out_specs=pl.BlockSpec((_N, 1), lambda i, k: (i, 0)),
        out_shape=jax.ShapeDtypeStruct((_P * _N, 1), jnp.float32),
        compiler_params=pltpu.CompilerParams(
            dimension_semantics=("parallel", "arbitrary"),
        ),
    )(t2d, x)


_L = 16


def _make_sc_gather(nw):
    bpw = _N // nw
    mesh = plsc.VectorSubcoreMesh(core_axis_name="c", subcore_axis_name="s")
    info = plsc.get_sparse_core_info()
    nc = info.num_cores

    @functools.partial(
        pl.kernel,
        mesh=mesh,
        out_type=jax.ShapeDtypeStruct((nw * _L,), jnp.float32),
        scratch_types=[
            pltpu.VMEM((bpw,), jnp.int32),
            pltpu.VMEM((bpw, 8, 128), jnp.float32),
            pltpu.VMEM((_L,), jnp.float32),
            pltpu.SemaphoreType.DMA,
        ],
    )
    def sc_gather(tgt_hbm, x_hbm, out_hbm, tgt_v, win_v, acc_v, sem):
        wid = lax.axis_index("s") * nc + lax.axis_index("c")
        base = wid * bpw
        pltpu.sync_copy(tgt_hbm.at[pl.ds(base, bpw)], tgt_v)
        lane = lax.iota(jnp.int32, _L)
        ts = []
        copies = []
        for j in range(bpw):
            tj = tgt_v[pl.ds((j // _L) * _L, _L)][j % _L]
            al = pl.multiple_of(
                lax.bitwise_and(tj, jnp.int32(~127)), 128
            )
            ts.append((tj, al))
            copies.append(
                pltpu.async_copy(
                    x_hbm.at[pl.ds(base + (j & ~7), 8), pl.ds(al, 128)],
                    win_v.at[j],
                    sem,
                )
            )
        for cp in copies:
            cp.wait()
        acc = jnp.zeros((_L,), jnp.float32)
        for j in range(bpw):
            tj, al = ts[j]
            rem = tj - al
            hi = pl.multiple_of(lax.shift_right_logical(rem, 4) * _L, _L)
            lo = lax.bitwise_and(rem, 15)
            w = win_v[j, j & 7, pl.ds(hi, _L)]
            lo = jnp.where(tj != 0, lo, jnp.int32(-1))
            acc = acc + jnp.where(lane == lo, w, 0.0)
        acc_v[...] = acc
        pltpu.sync_copy(acc_v, out_hbm.at[pl.ds(wid * _L, _L)])

    return sc_gather


def kernel(x, target):
    t32 = target.astype(jnp.int32)
    t2d = t32.reshape(_N, 1)
    tc_parts = _tc_reduce(x, t2d)

    info = plsc.get_sparse_core_info()
    nw = info.num_cores * info.num_subcores
    sc_parts = _make_sc_gather(nw)(t32, x)

    return jnp.sum(tc_parts) - jnp.float32(_CONF - _S) * jnp.sum(sc_parts)


# speedup vs baseline: 1.0177x; 1.0177x over previous
import functools
import math as _math

import jax
import jax.numpy as jnp
from jax import lax
from jax.experimental import pallas as pl
from jax.experimental.pallas import tpu as pltpu
from jax.experimental.pallas import tpu_sc as plsc

_SIZE = 100000
_N = 1024
_SMOOTHING = 0.1
_CONF = 1.0 - _SMOOTHING
_S = _SMOOTHING / (_SIZE - 2)
_C1 = _CONF * _math.log(_CONF) + _SMOOTHING * _math.log(_S)


_BC = 512
_P = 4
_NBC = 49
_NBTOT = _P * _NBC


def _tc_body(t_ref, x_ref, o_ref):
    i = pl.program_id(0)
    k = pl.program_id(1)
    gb = i * _NBC + k
    xb = x_ref[...]
    m = (t_ref[...] != 0).astype(jnp.float32)

    edge = jnp.logical_or(gb == 0, gb == _NBTOT - 1)

    @pl.when(k == 0)
    def _():
        o_ref[...] = jnp.zeros_like(o_ref)

    @pl.when(edge)
    def _():
        col = lax.broadcasted_iota(jnp.int32, xb.shape, 1) + gb * _BC
        valid = jnp.logical_and(col >= 1, col < _SIZE)
        rows = jnp.sum(jnp.where(valid, xb, 0.0), axis=1, keepdims=True)
        part = -_S * (rows * m)
        part = part + jnp.where(gb == 0, _C1 * m, 0.0)
        o_ref[...] += part

    @pl.when(jnp.logical_not(edge))
    def _():
        rows = jnp.sum(xb, axis=1, keepdims=True)
        o_ref[...] += -_S * (rows * m)


def _tc_reduce(x, t2d):
    return pl.pallas_call(
        _tc_body,
        grid=(_P, _NBC),
        in_specs=[
            pl.BlockSpec((_N, 1), lambda i, k: (0, 0)),
            pl.BlockSpec((_N, _BC), lambda i, k: (0, i * _NBC + k)),
        ],
        out_specs=pl.BlockSpec((_N, 1), lambda i, k: (i, 0)),
        out_shape=jax.ShapeDtypeStruct((_P * _N, 1), jnp.float32),
        compiler_params=pltpu.CompilerParams(
            dimension_semantics=("parallel", "arbitrary"),
        ),
    )(t2d, x)


_L = 16


def _make_sc_gather(nw):
    bpw = _N // nw
    mesh = plsc.VectorSubcoreMesh(core_axis_name="c", subcore_axis_name="s")
    info = plsc.get_sparse_core_info()
    nc = info.num_cores

    @functools.partial(
        pl.kernel,
        mesh=mesh,
        out_type=jax.ShapeDtypeStruct((nw * _L,), jnp.float32),
        scratch_types=[
            pltpu.VMEM((bpw,), jnp.int32),
            pltpu.VMEM((bpw, 8, 128), jnp.float32),
            pltpu.VMEM((_L,), jnp.float32),
            pltpu.SemaphoreType.DMA,
        ],
    )
    def sc_gather(tgt_hbm, x_hbm, out_hbm, tgt_v, win_v, acc_v, sem):
        wid = lax.axis_index("s") * nc + lax.axis_index("c")
        base = wid * bpw
        pltpu.sync_copy(tgt_hbm.at[pl.ds(base, bpw)], tgt_v)
        lane = lax.iota(jnp.int32, _L)
        ts = []
        copies = []
        for j in range(bpw):
            tj = tgt_v[pl.ds((j // _L) * _L, _L)][j % _L]
            al = pl.multiple_of(
                lax.bitwise_and(tj, jnp.int32(~127)), 128
            )
            ts.append((tj, al))
            copies.append(
                pltpu.async_copy(
                    x_hbm.at[pl.ds(base + (j & ~7), 8), pl.ds(al, 128)],
                    win_v.at[j],
                    sem,
                )
            )
        for cp in copies:
            cp.wait()
        acc = jnp.zeros((_L,), jnp.float32)
        for j in range(bpw):
            tj, al = ts[j]
            rem = tj - al
            hi = pl.multiple_of(lax.shift_right_logical(rem, 4) * _L, _L)
            lo = lax.bitwise_and(rem, 15)
            w = win_v[j, j & 7, pl.ds(hi, _L)]
            lo = jnp.where(tj != 0, lo, jnp.int32(-1))
            acc = acc + jnp.where(lane == lo, w, 0.0)
        acc_v[...] = acc
        pltpu.sync_copy(acc_v, out_hbm.at[pl.ds(wid * _L, _L)])

    return sc_gather


def kernel(x, target):
    t32 = target.astype(jnp.int32)
    t2d = t32.reshape(_N, 1)
    tc_parts = _tc_reduce(x, t2d)

    g = jnp.sum(jnp.where(t32 != 0, x[jnp.arange(_N), t32], 0.0))

    return jnp.sum(tc_parts) - jnp.float32(_CONF - _S) * g


# speedup vs baseline: 1.1743x; 1.1539x over previous
import functools
import math as _math

import jax
import jax.numpy as jnp
from jax import lax
from jax.experimental import pallas as pl
from jax.experimental.pallas import tpu as pltpu
from jax.experimental.pallas import tpu_sc as plsc

_SIZE = 100000
_N = 1024
_SMOOTHING = 0.1
_CONF = 1.0 - _SMOOTHING
_S = _SMOOTHING / (_SIZE - 2)
_C1 = _CONF * _math.log(_CONF) + _SMOOTHING * _math.log(_S)


_BC = 2560
_P = 2
_NBC = 20
_NBTOT = _P * _NBC


def _tc_body(t_ref, x_ref, o_ref):
    i = pl.program_id(0)
    k = pl.program_id(1)
    gb = i * _NBC + k
    xb = x_ref[...]
    m = (t_ref[...] != 0).astype(jnp.float32)

    edge = jnp.logical_or(gb == 0, gb == _NBTOT - 1)

    @pl.when(k == 0)
    def _():
        o_ref[...] = jnp.zeros_like(o_ref)

    @pl.when(edge)
    def _():
        col = lax.broadcasted_iota(jnp.int32, xb.shape, 1) + gb * _BC
        valid = jnp.logical_and(col >= 1, col < _SIZE)
        rows = jnp.sum(jnp.where(valid, xb, 0.0), axis=1, keepdims=True)
        part = -_S * (rows * m)
        part = part + jnp.where(gb == 0, _C1 * m, 0.0)
        o_ref[...] += part

    @pl.when(jnp.logical_not(edge))
    def _():
        rows = jnp.sum(xb, axis=1, keepdims=True)
        o_ref[...] += -_S * (rows * m)


def _tc_reduce(x, t2d):
    return pl.pallas_call(
        _tc_body,
        grid=(_P, _NBC),
        in_specs=[
            pl.BlockSpec((_N, 1), lambda i, k: (0, 0)),
            pl.BlockSpec((_N, _BC), lambda i, k: (0, i * _NBC + k)),
        ],
        out_specs=pl.BlockSpec((_N, 1), lambda i, k: (i, 0)),
        out_shape=jax.ShapeDtypeStruct((_P * _N, 1), jnp.float32),
        compiler_params=pltpu.CompilerParams(
            dimension_semantics=("parallel", "arbitrary"),
        ),
    )(t2d, x)


_L = 16


def _make_sc_gather(nw):
    bpw = _N // nw
    mesh = plsc.VectorSubcoreMesh(core_axis_name="c", subcore_axis_name="s")
    info = plsc.get_sparse_core_info()
    nc = info.num_cores

    @functools.partial(
        pl.kernel,
        mesh=mesh,
        out_type=jax.ShapeDtypeStruct((nw * _L,), jnp.float32),
        scratch_types=[
            pltpu.VMEM((bpw,), jnp.int32),
            pltpu.VMEM((bpw, 8, 128), jnp.float32),
            pltpu.VMEM((_L,), jnp.float32),
            pltpu.SemaphoreType.DMA,
        ],
    )
    def sc_gather(tgt_hbm, x_hbm, out_hbm, tgt_v, win_v, acc_v, sem):
        wid = lax.axis_index("s") * nc + lax.axis_index("c")
        base = wid * bpw
        pltpu.sync_copy(tgt_hbm.at[pl.ds(base, bpw)], tgt_v)
        lane = lax.iota(jnp.int32, _L)
        ts = []
        copies = []
        for j in range(bpw):
            tj = tgt_v[pl.ds((j // _L) * _L, _L)][j % _L]
            al = pl.multiple_of(
                lax.bitwise_and(tj, jnp.int32(~127)), 128
            )
            ts.append((tj, al))
            copies.append(
                pltpu.async_copy(
                    x_hbm.at[pl.ds(base + (j & ~7), 8), pl.ds(al, 128)],
                    win_v.at[j],
                    sem,
                )
            )
        for cp in copies:
            cp.wait()
        acc = jnp.zeros((_L,), jnp.float32)
        for j in range(bpw):
            tj, al = ts[j]
            rem = tj - al
            hi = pl.multiple_of(lax.shift_right_logical(rem, 4) * _L, _L)
            lo = lax.bitwise_and(rem, 15)
            w = win_v[j, j & 7, pl.ds(hi, _L)]
            lo = jnp.where(tj != 0, lo, jnp.int32(-1))
            acc = acc + jnp.where(lane == lo, w, 0.0)
        acc_v[...] = acc
        pltpu.sync_copy(acc_v, out_hbm.at[pl.ds(wid * _L, _L)])

    return sc_gather


def kernel(x, target):
    t32 = target.astype(jnp.int32)
    t2d = t32.reshape(_N, 1)
    tc_parts = _tc_reduce(x, t2d)

    info = plsc.get_sparse_core_info()
    nw = info.num_cores * info.num_subcores
    sc_parts = _make_sc_gather(nw)(t32, x)

    return jnp.sum(tc_parts) - jnp.float32(_CONF - _S) * jnp.sum(sc_parts)
